# baseline (speedup 1.0000x reference)
import functools

import jax
import jax.numpy as jnp
from jax import lax
from jax.experimental import pallas as pl
from jax.experimental.pallas import tpu as pltpu
from jax.experimental.pallas import tpu_sc as plsc

BATCH = 16384
DIM = 64
NCLASS = 100000
SPLIT = 49920
SEGB = NCLASS - SPLIT
SEGB_MAIN = 99840 - SPLIT
TAIL = NCLASS - 99840
LANES = 16
NW = 32
ROWS_PER_W = DIM // NW
FCHUNK = 8192
NFCH = BATCH // FCHUNK
GRP = 4


def _sc_loss_body(centers_hbm, featT_hbm, idx_hbm, out_hbm,
                  seg_a, seg_b, idx_v, feat_v, acc_v, seg_t,
                  sem_a, sem_b, sem_f):

    wid = lax.axis_index("s") * 2 + lax.axis_index("c")
    d0 = wid * ROWS_PER_W

    def copy_a(d):
        return pltpu.async_copy(centers_hbm.at[d, pl.ds(0, SPLIT)], seg_a,
                                sem_a)

    def copy_b(d):
        c1 = pltpu.async_copy(centers_hbm.at[d, pl.ds(SPLIT, SEGB_MAIN)],
                              seg_b, sem_b)
        c2 = pltpu.async_copy(centers_hbm.at[d, pl.ds(99840, TAIL)],
                              seg_t, sem_b)
        return c1, c2

    def mpass(base, upper, accs):
        @plsc.parallel_loop(0, FCHUNK // (LANES * GRP), unroll=2, carry=accs)
        def accs_out(g, acc_t):
            a = list(acc_t)
            for t in range(GRP):
                off = (g * GRP + t) * LANES
                iv = idx_v[pl.ds(base + off, LANES)]
                fv = feat_v[pl.ds(off, LANES)]
                if upper:
                    in_tail = iv >= 99840
                    m = jnp.logical_and(iv >= SPLIT,
                                        jnp.logical_not(in_tail))
                    gv = plsc.load_gather(seg_b, [iv - SPLIT], mask=m)
                    dv = fv - gv
                    acc = a[t] + jnp.where(m, dv * dv, 0.0)
                    gv2 = plsc.load_gather(seg_t, [iv - 99840], mask=in_tail)
                    dv2 = fv - gv2
                    a[t] = acc + jnp.where(in_tail, dv2 * dv2, 0.0)
                else:
                    m = iv < SPLIT
                    gv = plsc.load_gather(seg_a, [iv], mask=m)
                    dv = fv - gv
                    a[t] = a[t] + jnp.where(m, dv * dv, 0.0)
            return tuple(a)

        return accs_out

    cp_a = copy_a(d0)
    cp_b1, cp_b2 = copy_b(d0)
    pltpu.sync_copy(idx_hbm, idx_v)

    zeros = jnp.zeros((LANES,), jnp.float32)
    accs = (zeros, zeros, zeros, zeros)

    for r in range(ROWS_PER_W):
        d = d0 + r
        cp_f = pltpu.async_copy(featT_hbm.at[d, pl.ds(0, FCHUNK)], feat_v,
                                sem_f)
        cp_f.wait()
        cp_a.wait()
        accs = mpass(0, False, accs)
        cp_b1.wait()
        cp_b2.wait()
        accs = mpass(0, True, accs)
        pltpu.sync_copy(featT_hbm.at[d, pl.ds(FCHUNK, FCHUNK)], feat_v)
        accs = mpass(FCHUNK, False, accs)
        if r + 1 < ROWS_PER_W:
            cp_a = copy_a(d + 1)
        accs = mpass(FCHUNK, True, accs)
        if r + 1 < ROWS_PER_W:
            cp_b1, cp_b2 = copy_b(d + 1)

    acc_v[...] = accs[0] + accs[1] + accs[2] + accs[3]
    pltpu.sync_copy(acc_v, out_hbm.at[pl.ds(wid * LANES, LANES)])


_sc_loss = functools.partial(
    pl.kernel,
    out_type=jax.ShapeDtypeStruct((NW * LANES,), jnp.float32),
    mesh=plsc.VectorSubcoreMesh(core_axis_name="c", subcore_axis_name="s"),
    compiler_params=pltpu.CompilerParams(needs_layout_passes=False),
    scratch_types=[
        pltpu.VMEM((SPLIT,), jnp.float32),
        pltpu.VMEM((SEGB_MAIN,), jnp.float32),
        pltpu.VMEM((BATCH,), jnp.int32),
        pltpu.VMEM((FCHUNK,), jnp.float32),
        pltpu.VMEM((LANES,), jnp.float32),
        pltpu.VMEM((TAIL,), jnp.float32),
        pltpu.SemaphoreType.DMA,
        pltpu.SemaphoreType.DMA,
        pltpu.SemaphoreType.DMA,
    ],
)(_sc_loss_body)


def kernel(feat, centers, index):
    idx = index.astype(jnp.int32)
    partials = _sc_loss(centers, feat.T, idx)
    return jnp.sum(partials) * (1.0 / BATCH)


# speedup vs baseline: 1.1270x; 1.1270x over previous
import functools

import jax
import jax.numpy as jnp
from jax import lax
from jax.experimental import pallas as pl
from jax.experimental.pallas import tpu as pltpu
from jax.experimental.pallas import tpu_sc as plsc

BATCH = 16384
DIM = 64
NCLASS = 100000
LANES = 16
NW = 32
ROWS_PER_W = DIM // NW
FCHUNK = 8192
NFCH = BATCH // FCHUNK
GRP = 4


def _sc_loss_body(centers_hbm, featT_hbm, idx_hbm, out_hbm,
                  row_v, idx_v, feat_v, acc_v, sem_r, sem_f):
    wid = lax.axis_index("s") * 2 + lax.axis_index("c")

    zeros = jnp.zeros((LANES,), jnp.float32)

    def row_body(r, accs):
        d = wid * ROWS_PER_W + r
        cp = pltpu.async_copy(centers_hbm.at[d], row_v, sem_r)

        @pl.when(r == 0)
        def _():
            pltpu.sync_copy(idx_hbm, idx_v)

        cp_f = pltpu.async_copy(featT_hbm.at[d, pl.ds(0, FCHUNK)], feat_v,
                                sem_f)
        cp_f.wait()
        cp.wait()

        def chunk_body(c, accs2):
            base = c * FCHUNK

            @plsc.parallel_loop(0, FCHUNK // (LANES * GRP), unroll=2,
                                carry=accs2)
            def accs3(g, acc_t):
                a = list(acc_t)
                for t in range(GRP):
                    off = (g * GRP + t) * LANES
                    iv = idx_v[pl.ds(base + off, LANES)]
                    fv = feat_v[pl.ds(off, LANES)]
                    gv = plsc.load_gather(row_v, [iv])
                    dv = fv - gv
                    a[t] = a[t] + dv * dv
                return tuple(a)

            @pl.when(c + 1 < NFCH)
            def _():
                pltpu.sync_copy(
                    featT_hbm.at[d, pl.ds((c + 1) * FCHUNK, FCHUNK)], feat_v)

            return accs3

        return lax.fori_loop(0, NFCH, chunk_body, accs)

    accs = lax.fori_loop(0, ROWS_PER_W, row_body,
                         (zeros, zeros, zeros, zeros))
    acc_v[...] = accs[0] + accs[1] + accs[2] + accs[3]
    pltpu.sync_copy(acc_v, out_hbm.at[pl.ds(wid * LANES, LANES)])


_sc_loss = functools.partial(
    pl.kernel,
    out_type=jax.ShapeDtypeStruct((NW * LANES,), jnp.float32),
    mesh=plsc.VectorSubcoreMesh(core_axis_name="c", subcore_axis_name="s"),
    compiler_params=pltpu.CompilerParams(needs_layout_passes=False),
    scratch_types=[
        pltpu.VMEM((NCLASS,), jnp.float32),
        pltpu.VMEM((BATCH,), jnp.int32),
        pltpu.VMEM((FCHUNK,), jnp.float32),
        pltpu.VMEM((LANES,), jnp.float32),
        pltpu.SemaphoreType.DMA,
        pltpu.SemaphoreType.DMA,
    ],
)(_sc_loss_body)


def kernel(feat, centers, index):
    idx = index.astype(jnp.int32)
    partials = _sc_loss(centers, feat.T, idx)
    return jnp.sum(partials) * (1.0 / BATCH)
